# baseline (speedup 1.0000x reference)
import functools

import jax
import jax.numpy as jnp
from jax import lax
from jax.experimental import pallas as pl
from jax.experimental.pallas import tpu as pltpu
from jax.experimental.pallas import tpu_sc as plsc

MM = 10000
DD = 128
KK = 64
EE = 320000

NCORES = 2
NSUB = 16
NW = NCORES * NSUB
EDGES_PER_W = EE // NW
BLK = 8
NBLK = EDGES_PER_W // BLK
CHUNK = 50
NCHUNK = NBLK // CHUNK
ROW_W = 656
ACC_W = 144
ROWS_PER_SUB = MM // NSUB


def _prep_body(x_ref, mem_ref,
               vcx, vcm, wcx, wcm,
               vss, wss, vso, wso, wo_r,
               vos, wos, voo, woo, ws_r,
               s_ref, o_ref):
    x = x_ref[...]
    mem = mem_ref[...]
    f32 = jnp.float32
    g = jax.nn.sigmoid(jnp.dot(x, vcx[...], preferred_element_type=f32)
                       + jnp.dot(mem, vcm[...], preferred_element_type=f32))
    d = jax.nn.relu(jnp.dot(x, wcx[...], preferred_element_type=f32)
                    + jnp.dot(mem, wcm[...], preferred_element_type=f32))
    v = x + g * d
    n = jnp.sqrt(jnp.sum(v * v, axis=1, keepdims=True))
    x1 = v / jnp.maximum(n, 1e-12)

    s_ref[:, 0:128] = x1
    s_ref[:, 128:256] = jnp.dot(x1, vss[...], preferred_element_type=f32)
    s_ref[:, 256:384] = jnp.dot(x1, wss[...], preferred_element_type=f32)
    s_ref[:, 384:512] = jnp.dot(x1, vso[...], preferred_element_type=f32)
    s_ref[:, 512:640] = jnp.dot(x1, wso[...], preferred_element_type=f32)
    ssc = jnp.sum(x1 * wo_r[...], axis=1, keepdims=True)
    s_ref[:, 640:656] = jnp.broadcast_to(ssc, (ssc.shape[0], 16))

    o_ref[:, 0:128] = x1
    o_ref[:, 128:256] = jnp.dot(x1, vos[...], preferred_element_type=f32)
    o_ref[:, 256:384] = jnp.dot(x1, wos[...], preferred_element_type=f32)
    o_ref[:, 384:512] = jnp.dot(x1, voo[...], preferred_element_type=f32)
    o_ref[:, 512:640] = jnp.dot(x1, woo[...], preferred_element_type=f32)
    osc = jnp.sum(x1 * ws_r[...], axis=1, keepdims=True)
    o_ref[:, 640:656] = jnp.broadcast_to(osc, (osc.shape[0], 16))


def _prep_tables(x, mem, Vc, Wc, Vs_s, Vo_s, Ws_s, Wo_s, w_s,
                 Vs_o, Vo_o, Ws_o, Wo_o, w_o):
    bm = 1000
    grid = MM // bm
    full = lambda a: pl.BlockSpec(a.shape, lambda i: (0,) * a.ndim)
    vcx, vcm = Vc.T[:DD], Vc.T[DD:]
    wcx, wcm = Wc.T[:DD], Wc.T[DD:]
    wo_r = w_o.reshape(1, DD)
    ws_r = w_s.reshape(1, DD)
    weights = (vcx, vcm, wcx, wcm,
               Vs_s.T, Ws_s.T, Vs_o.T, Ws_o.T, wo_r,
               Vo_s.T, Wo_s.T, Vo_o.T, Wo_o.T, ws_r)
    return pl.pallas_call(
        _prep_body,
        grid=(grid,),
        in_specs=[pl.BlockSpec((bm, DD), lambda i: (i, 0)),
                  pl.BlockSpec((bm, KK), lambda i: (i, 0))]
                 + [full(w) for w in weights],
        out_specs=[pl.BlockSpec((bm, ROW_W), lambda i: (i, 0)),
                   pl.BlockSpec((bm, ROW_W), lambda i: (i, 0))],
        out_shape=[jax.ShapeDtypeStruct((MM, ROW_W), jnp.float32),
                   jax.ShapeDtypeStruct((MM, ROW_W), jnp.float32)],
    )(x, mem, *weights)


def _rsqrt16(ssum):
    total = jnp.maximum(jnp.sum(ssum), 1e-24)
    tv = jnp.full((16,), total, dtype=jnp.float32)
    bits = plsc.bitcast(tv, jnp.int32)
    y = plsc.bitcast(jnp.int32(0x5F3759DF) - (bits >> 1), jnp.float32)
    for _ in range(3):
        y = y * (1.5 - 0.5 * tv * y * y)
    return y


def _sigmoid16(z):
    return 1.0 / (1.0 + jnp.exp(-z))


def _sc_body(s_hbm, o_hbm, idx_hbm, out_hbm,
             chunk_i, s_rows, o_rows, upd, zbuf, acc,
             gsem0, gsem1, gsem2, ssem0, ssem1):
    cid = lax.axis_index("c")
    sid = lax.axis_index("s")
    wid = sid * NCORES + cid
    f32 = jnp.float32
    zeros16 = jnp.zeros((16,), f32)
    ones16 = jnp.ones((16,), f32)

    def zrow(r, _):
        for c in range(ACC_W // 16):
            zbuf[r, pl.ds(c * 16, 16)] = zeros16
        return 0
    lax.fori_loop(0, 5, zrow, 0)

    def zchunk(k, _):
        pltpu.sync_copy(zbuf, acc.at[pl.ds(sid * ROWS_PER_SUB + k * 5, 5)])
        return 0
    lax.fori_loop(0, ROWS_PER_SUB // 5, zchunk, 0)
    plsc.subcore_barrier()

    def make_edge(p, q):
        def edge(i, _):
            sc_s = s_rows[p, i, pl.ds(640, 16)]
            sc_o = o_rows[p, i, pl.ds(640, 16)]

            ts = []
            ssum = zeros16
            for c in range(8):
                off = c * 16
                xs = s_rows[p, i, pl.ds(off, 16)]
                gate = _sigmoid16(s_rows[p, i, pl.ds(128 + off, 16)]
                                  + o_rows[p, i, pl.ds(128 + off, 16)])
                dirv = jnp.maximum(s_rows[p, i, pl.ds(256 + off, 16)]
                                   + o_rows[p, i, pl.ds(256 + off, 16)]
                                   + xs * sc_o, 0.0)
                t = xs + gate * dirv
                ts.append(t)
                ssum = ssum + t * t
            inv = _rsqrt16(ssum)
            for c in range(8):
                upd[q, i, pl.ds(c * 16, 16)] = ts[c] * inv
            upd[q, i, pl.ds(128, 16)] = ones16

            to = []
            osum = zeros16
            for c in range(8):
                off = c * 16
                xo = o_rows[p, i, pl.ds(off, 16)]
                gate = _sigmoid16(s_rows[p, i, pl.ds(384 + off, 16)]
                                  + o_rows[p, i, pl.ds(384 + off, 16)])
                dirv = jnp.maximum(s_rows[p, i, pl.ds(512 + off, 16)]
                                   + o_rows[p, i, pl.ds(512 + off, 16)]
                                   + xo * sc_s, 0.0)
                t = xo + gate * dirv
                to.append(t)
                osum = osum + t * t
            inv = _rsqrt16(osum)
            for c in range(8):
                upd[q, BLK + i, pl.ds(c * 16, 16)] = to[c] * inv
            upd[q, BLK + i, pl.ds(128, 16)] = ones16
            return 0
        return edge

    gsems = (gsem0, gsem1, gsem2)
    ssems = (ssem0, ssem1)

    def fire_gather(j, p, gsem):
        pltpu.async_copy(
            s_hbm.at[chunk_i.at[j, pl.ds(0, BLK)]], s_rows.at[p], gsem)
        pltpu.async_copy(
            o_hbm.at[chunk_i.at[j, pl.ds(BLK, BLK)]], o_rows.at[p], gsem)

    def drain_gather(gsem):
        pltpu.make_async_copy(s_hbm.at[chunk_i.at[0, pl.ds(0, BLK)]],
                              s_rows.at[0], gsem).wait()
        pltpu.make_async_copy(o_hbm.at[chunk_i.at[0, pl.ds(0, BLK)]],
                              o_rows.at[0], gsem).wait()

    def drain_scatter(ssem):
        pltpu.make_async_copy(upd.at[0], acc.at[chunk_i.at[0]], ssem).wait()

    wrow = wid * NBLK

    def block(g, _):
        j = g % CHUNK
        c = g // CHUNK
        p = g % 3
        q = g % 2

        @pl.when(j == 0)
        def _():
            @pl.when(g >= 2)
            def _():
                drain_scatter(ssems[0])
                drain_scatter(ssems[1])
            pltpu.sync_copy(idx_hbm.at[pl.ds(wrow + c * CHUNK, CHUNK)],
                            chunk_i)
            for d in range(3):
                pd = (g + d) % 3
                for t in range(3):
                    @pl.when(pd == t)
                    def _(d=d, t=t):
                        fire_gather(d, t, gsems[t])

        @pl.when(jnp.logical_and(j > 0, j + 2 < CHUNK))
        def _():
            for t in range(3):
                @pl.when((g + 2) % 3 == t)
                def _(t=t):
                    fire_gather(j + 2, t, gsems[t])

        for t in range(3):
            @pl.when(p == t)
            def _(t=t):
                drain_gather(gsems[t])

        @pl.when(jnp.logical_and(g >= 2, j >= 2))
        def _():
            for t in range(2):
                @pl.when(q == t)
                def _(t=t):
                    drain_scatter(ssems[t])

        lax.fori_loop(0, BLK, make_edge(p, q), 0, unroll=2)

        for t in range(2):
            @pl.when(q == t)
            def _(t=t):
                pltpu.async_copy(upd.at[q], acc.at[chunk_i.at[j]],
                                 ssems[t], add=True)
        return 0

    lax.fori_loop(0, NBLK, block, 0)
    drain_scatter(ssem0)
    drain_scatter(ssem1)
    plsc.subcore_barrier()

    r0 = sid * ROWS_PER_SUB

    @pl.when(cid == 0)
    def _():
        pltpu.sync_copy(acc.at[pl.ds(r0, ROWS_PER_SUB)],
                        out_hbm.at[0, pl.ds(r0, ROWS_PER_SUB)])

    @pl.when(cid == 1)
    def _():
        pltpu.sync_copy(acc.at[pl.ds(r0, ROWS_PER_SUB)],
                        out_hbm.at[1, pl.ds(r0, ROWS_PER_SUB)])


def _sc_stage(s_table, o_table, idx16):
    mesh = plsc.VectorSubcoreMesh(core_axis_name="c", subcore_axis_name="s")
    fn = functools.partial(
        pl.kernel,
        mesh=mesh,
        compiler_params=pltpu.CompilerParams(use_tc_tiling_on_sc=False,
                                             needs_layout_passes=False),
        out_type=jax.ShapeDtypeStruct((NCORES, MM, ACC_W), jnp.float32),
        scratch_types=[
            pltpu.VMEM((CHUNK, 16), jnp.int32),
            pltpu.VMEM((3, BLK, ROW_W), jnp.float32),
            pltpu.VMEM((3, BLK, ROW_W), jnp.float32),
            pltpu.VMEM((2, 2 * BLK, ACC_W), jnp.float32),
            pltpu.VMEM((5, ACC_W), jnp.float32),
            pltpu.VMEM_SHARED((MM, ACC_W), jnp.float32),
            pltpu.SemaphoreType.DMA,
            pltpu.SemaphoreType.DMA,
            pltpu.SemaphoreType.DMA,
            pltpu.SemaphoreType.DMA,
            pltpu.SemaphoreType.DMA,
        ],
    )(_sc_body)
    return fn(s_table, o_table, idx16)


def _fin_body(ob_ref, s_ref, out_ref):
    a = ob_ref[0][:, 0:128] + ob_ref[1][:, 0:128]
    c16 = ob_ref[0][:, 128:144] + ob_ref[1][:, 128:144]
    cnt = jnp.max(c16, axis=1, keepdims=True)
    x1 = s_ref[...]
    avg = jnp.where(cnt > 0, a / jnp.maximum(cnt, 1.0), x1)
    n = jnp.sqrt(jnp.sum(avg * avg, axis=1, keepdims=True))
    out_ref[...] = avg / jnp.maximum(n, 1e-12)


def _finish(acc2, s_table):
    bm = 1000
    grid = MM // bm
    return pl.pallas_call(
        _fin_body,
        grid=(grid,),
        in_specs=[pl.BlockSpec((NCORES, bm, ACC_W), lambda i: (0, i, 0)),
                  pl.BlockSpec((bm, DD), lambda i: (i, 0))],
        out_specs=pl.BlockSpec((bm, DD), lambda i: (i, 0)),
        out_shape=jax.ShapeDtypeStruct((MM, DD), jnp.float32),
    )(acc2, s_table)


def kernel(individual_embeddings, individual_memberships, edge_index,
           Vc, Wc,
           Vs_s, Vo_s, Ws_s, Wo_s, w_s,
           Vs_o, Vo_o, Ws_o, Wo_o, w_o):
    idx16 = jnp.concatenate(
        [edge_index[0].astype(jnp.int32).reshape(EE // BLK, BLK),
         edge_index[1].astype(jnp.int32).reshape(EE // BLK, BLK)], axis=1)
    s_table, o_table = _prep_tables(
        individual_embeddings, individual_memberships,
        Vc, Wc, Vs_s, Vo_s, Ws_s, Wo_s, w_s, Vs_o, Vo_o, Ws_o, Wo_o, w_o)
    acc2 = _sc_stage(s_table, o_table, idx16)
    return _finish(acc2, s_table)


# speedup vs baseline: 1.5221x; 1.5221x over previous
import functools

import jax
import jax.numpy as jnp
from jax import lax
from jax.experimental import pallas as pl
from jax.experimental.pallas import tpu as pltpu
from jax.experimental.pallas import tpu_sc as plsc

MM = 10000
DD = 128
KK = 64
EE = 320000

NCORES = 2
NSUB = 16
NW = NCORES * NSUB
EDGES_PER_W = EE // NW
BLK = 8
NBLK = EDGES_PER_W // BLK
CHUNK = 250
NCHUNK = NBLK // CHUNK
ROW_W = 656
ACC_W = 144
ROWS_PER_SUB = MM // NSUB


def _prep_body(x_ref, mem_ref,
               vcx, vcm, wcx, wcm,
               vss, wss, vso, wso, wo_r,
               vos, wos, voo, woo, ws_r,
               s_ref, o_ref):
    x = x_ref[...]
    mem = mem_ref[...]
    f32 = jnp.float32
    g = jax.nn.sigmoid(jnp.dot(x, vcx[...], preferred_element_type=f32)
                       + jnp.dot(mem, vcm[...], preferred_element_type=f32))
    d = jax.nn.relu(jnp.dot(x, wcx[...], preferred_element_type=f32)
                    + jnp.dot(mem, wcm[...], preferred_element_type=f32))
    v = x + g * d
    n = jnp.sqrt(jnp.sum(v * v, axis=1, keepdims=True))
    x1 = v / jnp.maximum(n, 1e-12)

    s_ref[:, 0:128] = x1
    s_ref[:, 128:256] = jnp.dot(x1, vss[...], preferred_element_type=f32)
    s_ref[:, 256:384] = jnp.dot(x1, wss[...], preferred_element_type=f32)
    s_ref[:, 384:512] = jnp.dot(x1, vso[...], preferred_element_type=f32)
    s_ref[:, 512:640] = jnp.dot(x1, wso[...], preferred_element_type=f32)
    ssc = jnp.sum(x1 * wo_r[...], axis=1, keepdims=True)
    s_ref[:, 640:656] = jnp.broadcast_to(ssc, (ssc.shape[0], 16))

    o_ref[:, 0:128] = x1
    o_ref[:, 128:256] = jnp.dot(x1, vos[...], preferred_element_type=f32)
    o_ref[:, 256:384] = jnp.dot(x1, wos[...], preferred_element_type=f32)
    o_ref[:, 384:512] = jnp.dot(x1, voo[...], preferred_element_type=f32)
    o_ref[:, 512:640] = jnp.dot(x1, woo[...], preferred_element_type=f32)
    osc = jnp.sum(x1 * ws_r[...], axis=1, keepdims=True)
    o_ref[:, 640:656] = jnp.broadcast_to(osc, (osc.shape[0], 16))


def _prep_tables(x, mem, Vc, Wc, Vs_s, Vo_s, Ws_s, Wo_s, w_s,
                 Vs_o, Vo_o, Ws_o, Wo_o, w_o):
    bm = 1000
    grid = MM // bm
    full = lambda a: pl.BlockSpec(a.shape, lambda i: (0,) * a.ndim)
    vcx, vcm = Vc.T[:DD], Vc.T[DD:]
    wcx, wcm = Wc.T[:DD], Wc.T[DD:]
    wo_r = w_o.reshape(1, DD)
    ws_r = w_s.reshape(1, DD)
    weights = (vcx, vcm, wcx, wcm,
               Vs_s.T, Ws_s.T, Vs_o.T, Ws_o.T, wo_r,
               Vo_s.T, Wo_s.T, Vo_o.T, Wo_o.T, ws_r)
    return pl.pallas_call(
        _prep_body,
        grid=(grid,),
        in_specs=[pl.BlockSpec((bm, DD), lambda i: (i, 0)),
                  pl.BlockSpec((bm, KK), lambda i: (i, 0))]
                 + [full(w) for w in weights],
        out_specs=[pl.BlockSpec((bm, ROW_W), lambda i: (i, 0)),
                   pl.BlockSpec((bm, ROW_W), lambda i: (i, 0))],
        out_shape=[jax.ShapeDtypeStruct((MM, ROW_W), jnp.float32),
                   jax.ShapeDtypeStruct((MM, ROW_W), jnp.float32)],
    )(x, mem, *weights)


def _rsqrt16(ssum):
    total = jnp.maximum(jnp.sum(ssum), 1e-24)
    tv = jnp.full((16,), total, dtype=jnp.float32)
    bits = plsc.bitcast(tv, jnp.int32)
    y = plsc.bitcast(jnp.int32(0x5F3759DF) - (bits >> 1), jnp.float32)
    for _ in range(2):
        y = y * (1.5 - 0.5 * tv * y * y)
    return y


def _sigmoid16(z):
    return 1.0 / (1.0 + jnp.exp(-z))


def _sc_body(s_hbm, o_hbm, idx_hbm, out_hbm,
             chunk_i, s_rows, o_rows, upd, zbuf, acc,
             gsem0, gsem1, gsem2, ssem0, ssem1):
    cid = lax.axis_index("c")
    sid = lax.axis_index("s")
    wid = sid * NCORES + cid
    f32 = jnp.float32
    zeros16 = jnp.zeros((16,), f32)
    ones16 = jnp.ones((16,), f32)

    def zrow(r, _):
        for c in range(ACC_W // 16):
            zbuf[r, pl.ds(c * 16, 16)] = zeros16
        return 0
    lax.fori_loop(0, 5, zrow, 0)

    def zchunk(k, _):
        pltpu.sync_copy(zbuf, acc.at[pl.ds(sid * ROWS_PER_SUB + k * 5, 5)])
        return 0
    lax.fori_loop(0, ROWS_PER_SUB // 5, zchunk, 0)
    plsc.subcore_barrier()

    def make_edge(p, q):
        def edge(i, _):
            sc_s = s_rows[p, i, pl.ds(640, 16)]
            sc_o = o_rows[p, i, pl.ds(640, 16)]

            ts = []
            ssum = zeros16
            for c in range(8):
                off = c * 16
                xs = s_rows[p, i, pl.ds(off, 16)]
                gate = _sigmoid16(s_rows[p, i, pl.ds(128 + off, 16)]
                                  + o_rows[p, i, pl.ds(128 + off, 16)])
                dirv = jnp.maximum(s_rows[p, i, pl.ds(256 + off, 16)]
                                   + o_rows[p, i, pl.ds(256 + off, 16)]
                                   + xs * sc_o, 0.0)
                t = xs + gate * dirv
                ts.append(t)
                ssum = ssum + t * t
            inv = _rsqrt16(ssum)
            for c in range(8):
                upd[q, i, pl.ds(c * 16, 16)] = ts[c] * inv
            upd[q, i, pl.ds(128, 16)] = ones16

            to = []
            osum = zeros16
            for c in range(8):
                off = c * 16
                xo = o_rows[p, i, pl.ds(off, 16)]
                gate = _sigmoid16(s_rows[p, i, pl.ds(384 + off, 16)]
                                  + o_rows[p, i, pl.ds(384 + off, 16)])
                dirv = jnp.maximum(s_rows[p, i, pl.ds(512 + off, 16)]
                                   + o_rows[p, i, pl.ds(512 + off, 16)]
                                   + xo * sc_s, 0.0)
                t = xo + gate * dirv
                to.append(t)
                osum = osum + t * t
            inv = _rsqrt16(osum)
            for c in range(8):
                upd[q, BLK + i, pl.ds(c * 16, 16)] = to[c] * inv
            upd[q, BLK + i, pl.ds(128, 16)] = ones16
            return 0
        return edge

    gsems = (gsem0, gsem1, gsem2)
    ssems = (ssem0, ssem1)

    def fire_gather(j, p, gsem):
        pltpu.async_copy(
            s_hbm.at[chunk_i.at[j, pl.ds(0, BLK)]], s_rows.at[p], gsem)
        pltpu.async_copy(
            o_hbm.at[chunk_i.at[j, pl.ds(BLK, BLK)]], o_rows.at[p], gsem)

    def drain_gather(gsem):
        pltpu.make_async_copy(s_hbm.at[chunk_i.at[0, pl.ds(0, BLK)]],
                              s_rows.at[0], gsem).wait()
        pltpu.make_async_copy(o_hbm.at[chunk_i.at[0, pl.ds(0, BLK)]],
                              o_rows.at[0], gsem).wait()

    def drain_scatter(ssem):
        pltpu.make_async_copy(upd.at[0], acc.at[chunk_i.at[0]], ssem).wait()

    wrow = wid * NBLK

    def block(g, _):
        j = g % CHUNK
        c = g // CHUNK
        p = g % 3
        q = g % 2

        @pl.when(j == 0)
        def _():
            @pl.when(g >= 2)
            def _():
                drain_scatter(ssems[0])
                drain_scatter(ssems[1])
            pltpu.sync_copy(idx_hbm.at[pl.ds(wrow + c * CHUNK, CHUNK)],
                            chunk_i)
            for d in range(3):
                pd = (g + d) % 3
                for t in range(3):
                    @pl.when(pd == t)
                    def _(d=d, t=t):
                        fire_gather(d, t, gsems[t])

        @pl.when(jnp.logical_and(j > 0, j + 2 < CHUNK))
        def _():
            for t in range(3):
                @pl.when((g + 2) % 3 == t)
                def _(t=t):
                    fire_gather(j + 2, t, gsems[t])

        for t in range(3):
            @pl.when(p == t)
            def _(t=t):
                drain_gather(gsems[t])

        @pl.when(jnp.logical_and(g >= 2, j >= 2))
        def _():
            for t in range(2):
                @pl.when(q == t)
                def _(t=t):
                    drain_scatter(ssems[t])

        lax.fori_loop(0, BLK, make_edge(p, q), 0)

        for t in range(2):
            @pl.when(q == t)
            def _(t=t):
                pltpu.async_copy(upd.at[q], acc.at[chunk_i.at[j]],
                                 ssems[t], add=True)
        return 0

    lax.fori_loop(0, NBLK, block, 0)
    drain_scatter(ssem0)
    drain_scatter(ssem1)
    plsc.subcore_barrier()

    r0 = sid * ROWS_PER_SUB

    @pl.when(cid == 0)
    def _():
        pltpu.sync_copy(acc.at[pl.ds(r0, ROWS_PER_SUB)],
                        out_hbm.at[0, pl.ds(r0, ROWS_PER_SUB)])

    @pl.when(cid == 1)
    def _():
        pltpu.sync_copy(acc.at[pl.ds(r0, ROWS_PER_SUB)],
                        out_hbm.at[1, pl.ds(r0, ROWS_PER_SUB)])


def _sc_stage(s_table, o_table, idx16):
    mesh = plsc.VectorSubcoreMesh(core_axis_name="c", subcore_axis_name="s")
    fn = functools.partial(
        pl.kernel,
        mesh=mesh,
        compiler_params=pltpu.CompilerParams(use_tc_tiling_on_sc=False,
                                             needs_layout_passes=False),
        out_type=jax.ShapeDtypeStruct((NCORES, MM, ACC_W), jnp.float32),
        scratch_types=[
            pltpu.VMEM((CHUNK, 16), jnp.int32),
            pltpu.VMEM((3, BLK, ROW_W), jnp.float32),
            pltpu.VMEM((3, BLK, ROW_W), jnp.float32),
            pltpu.VMEM((2, 2 * BLK, ACC_W), jnp.float32),
            pltpu.VMEM((5, ACC_W), jnp.float32),
            pltpu.VMEM_SHARED((MM, ACC_W), jnp.float32),
            pltpu.SemaphoreType.DMA,
            pltpu.SemaphoreType.DMA,
            pltpu.SemaphoreType.DMA,
            pltpu.SemaphoreType.DMA,
            pltpu.SemaphoreType.DMA,
        ],
    )(_sc_body)
    return fn(s_table, o_table, idx16)


def _fin_body(ob_ref, s_ref, out_ref):
    a = ob_ref[0][:, 0:128] + ob_ref[1][:, 0:128]
    c16 = ob_ref[0][:, 128:144] + ob_ref[1][:, 128:144]
    cnt = jnp.max(c16, axis=1, keepdims=True)
    x1 = s_ref[...]
    avg = jnp.where(cnt > 0, a / jnp.maximum(cnt, 1.0), x1)
    n = jnp.sqrt(jnp.sum(avg * avg, axis=1, keepdims=True))
    out_ref[...] = avg / jnp.maximum(n, 1e-12)


def _finish(acc2, s_table):
    bm = 1000
    grid = MM // bm
    return pl.pallas_call(
        _fin_body,
        grid=(grid,),
        in_specs=[pl.BlockSpec((NCORES, bm, ACC_W), lambda i: (0, i, 0)),
                  pl.BlockSpec((bm, DD), lambda i: (i, 0))],
        out_specs=pl.BlockSpec((bm, DD), lambda i: (i, 0)),
        out_shape=jax.ShapeDtypeStruct((MM, DD), jnp.float32),
    )(acc2, s_table)


def kernel(individual_embeddings, individual_memberships, edge_index,
           Vc, Wc,
           Vs_s, Vo_s, Ws_s, Wo_s, w_s,
           Vs_o, Vo_o, Ws_o, Wo_o, w_o):
    idx16 = jnp.concatenate(
        [edge_index[0].astype(jnp.int32).reshape(EE // BLK, BLK),
         edge_index[1].astype(jnp.int32).reshape(EE // BLK, BLK)], axis=1)
    s_table, o_table = _prep_tables(
        individual_embeddings, individual_memberships,
        Vc, Wc, Vs_s, Vo_s, Ws_s, Wo_s, w_s, Vs_o, Vo_o, Ws_o, Wo_o, w_o)
    acc2 = _sc_stage(s_table, o_table, idx16)
    return _finish(acc2, s_table)
